# baseline (speedup 1.0000x reference)
import functools

import jax
import jax.numpy as jnp
from jax import lax
from jax.experimental import pallas as pl
from jax.experimental.pallas import tpu as pltpu
from jax.experimental.pallas import tpu_sc as plsc

EMBED_DIM = 64
BATCH = 16384
NC = 2
NS = 16
NW = NC * NS
BPW = BATCH // NW
CHUNK = 16
NCHUNK = BPW // CHUNK
NBUF = 2
NROWS = 1000000
NBLK = (NROWS + 127) // 128
BLKW = 248
NBATCH_I = 2 * BATCH
CAP_U = 1536
CAP_I = 3072
RING = 16


def _scan_body(ut, it, lu_hbm, li_hbm, su_hbm, si_hbm,
               scr_u, scr_i,
               su_v, si_v, lu_v, li_v,
               slab_u, slab_i, ring_u, ring_i,
               semb0, semb1, semr_u, semr_i):
    c = lax.axis_index("c")
    s = lax.axis_index("s")
    w = s * NC + c
    q0 = w * BLKW

    pltpu.sync_copy(su_hbm.at[pl.ds(q0, 272)], su_v)
    pltpu.sync_copy(si_hbm.at[pl.ds(q0, 272)], si_v)

    s_u = su_v[pl.ds(0, 16)][0]
    a_u = pl.multiple_of((s_u // 8) * 8, 8)
    pltpu.sync_copy(lu_hbm.at[pl.ds(a_u, CAP_U)], lu_v)
    s_i = si_v[pl.ds(0, 16)][0]
    a_i = pl.multiple_of((s_i // 8) * 8, 8)
    pltpu.sync_copy(li_hbm.at[pl.ds(a_i, CAP_I)], li_v)

    sembs = [semb0, semb1]
    dims = [lax.iota(jnp.int32, 16) + 16 * cc for cc in range(4)]

    def fire(j, buf, in_range):
        q = q0 + j

        @pl.when(in_range & (q < NBLK))
        def _():
            off = pl.multiple_of(q * 128, 128)
            pltpu.async_copy(ut.at[:, pl.ds(off, 128)], slab_u.at[buf],
                             sembs[buf])
            pltpu.async_copy(it.at[:, pl.ds(off, 128)], slab_i.at[buf],
                             sembs[buf])

    def wait_slabs(j, buf):
        @pl.when(q0 + j < NBLK)
        def _():
            pltpu.make_async_copy(ut.at[:, pl.ds(0, 128)], slab_u.at[buf],
                                  sembs[buf]).wait()
            pltpu.make_async_copy(it.at[:, pl.ds(0, 128)], slab_i.at[buf],
                                  sembs[buf]).wait()

    fire(0, 0, True)

    def do_block(j, buf, carry):
        h_u, h_i = carry

        fire(j + 1, (buf + 1) % NBUF, j + 1 < BLKW)
        wait_slabs(j, buf)

        def handle(slab, lanes_v, a0, starts_v, ring, semr, scr, h0):
            p0 = starts_v[pl.ds(j, 16)][0]
            p1 = starts_v[pl.ds(j + 1, 16)][0]

            def hit(p, h):
                l = lanes_v[pl.ds(p - a0, 16)][0]
                lv = jnp.full((16,), l, jnp.int32)
                r = h % RING

                @pl.when(h >= RING)
                def _():
                    pltpu.make_async_copy(scr.at[0], ring.at[0], semr).wait()

                for cc in range(4):
                    ring[r, pl.ds(16 * cc, 16)] = plsc.load_gather(
                        slab.at[buf], [dims[cc], lv])
                pltpu.async_copy(ring.at[r], scr.at[p], semr)
                return h + 1

            return lax.fori_loop(p0, p1, hit, h0)

        h_u = handle(slab_u, lu_v, a_u, su_v, ring_u, semr_u, scr_u, h_u)
        h_i = handle(slab_i, li_v, a_i, si_v, ring_i, semr_i, scr_i, h_i)
        return h_u, h_i

    def block_pair(jj, carry):
        for buf in range(NBUF):
            carry = do_block(jj * NBUF + buf, buf, carry)
        return carry

    h_u, h_i = lax.fori_loop(0, BLKW // NBUF, block_pair,
                             (jnp.int32(0), jnp.int32(0)))

    for i in range(RING):
        @pl.when(i < h_u)
        def _():
            pltpu.make_async_copy(scr_u.at[0], ring_u.at[0], semr_u).wait()

        @pl.when(i < h_i)
        def _():
            pltpu.make_async_copy(scr_i.at[0], ring_i.at[0], semr_i).wait()


def _compute_body(scr_u, scr_i, w_hbm, b_hbm, pu_hbm, pp_hbm, pn_hbm,
                  pos_hbm, neg_hbm,
                  pu_v, pp_v, pn_v, w_v, b_v,
                  u_buf, p_buf, n_buf, tmp_p, tmp_n, outp_v, outn_v,
                  sem0, sem1):
    c = lax.axis_index("c")
    s = lax.axis_index("s")
    wid = s * NC + c
    base = wid * BPW

    pltpu.sync_copy(pu_hbm.at[pl.ds(base, BPW)], pu_v)
    pltpu.sync_copy(pp_hbm.at[pl.ds(base, BPW)], pp_v)
    pltpu.sync_copy(pn_hbm.at[pl.ds(base, BPW)], pn_v)
    pltpu.sync_copy(w_hbm, w_v)
    pltpu.sync_copy(b_hbm, b_v)

    sems = [sem0, sem1]
    bufs = [(u_buf, pu_v, scr_u), (p_buf, pp_v, scr_i), (n_buf, pn_v, scr_i)]

    def fire(k, slot):
        off = pl.multiple_of(k * CHUNK, CHUNK)
        for buf, idx_v, table in bufs:
            iv = idx_v[pl.ds(off, CHUNK)]
            for j in range(CHUNK):
                pltpu.async_copy(table.at[iv[j]], buf.at[slot, j], sems[slot])

    def drain(slot):
        for buf, idx_v, table in bufs:
            pltpu.make_async_copy(table.at[pl.ds(0, CHUNK)], buf.at[slot],
                                  sems[slot]).wait()

    lanes = lax.iota(jnp.int32, 16)
    last = lanes == 15
    wc = [w_v[pl.ds(cc * 16, 16)] for cc in range(EMBED_DIM // 16)]
    bvec = b_v[...]

    fire(0, 0)

    def step(kk, carry):
        for slot in range(NBUF):
            k = kk * NBUF + slot
            nxt = (slot + 1) % NBUF

            @pl.when(k + 1 < NCHUNK)
            def _():
                fire(k + 1, nxt)

            drain(slot)

            off = pl.multiple_of(k * CHUNK, CHUNK)
            for j in range(CHUNK):
                accp = None
                accn = None
                for cc in range(EMBED_DIM // 16):
                    sl = pl.ds(cc * 16, 16)
                    uw = u_buf[slot, j, sl] * wc[cc]
                    tp = uw * p_buf[slot, j, sl]
                    tn = uw * n_buf[slot, j, sl]
                    accp = tp if accp is None else accp + tp
                    accn = tn if accn is None else accn + tn
                idxj = jnp.full((16,), j, jnp.int32)
                plsc.store_scatter(tmp_p, [idxj], plsc.cumsum(accp), mask=last)
                plsc.store_scatter(tmp_n, [idxj], plsc.cumsum(accn), mask=last)
            outp_v[pl.ds(off, CHUNK)] = tmp_p[...] + bvec
            outn_v[pl.ds(off, CHUNK)] = tmp_n[...] + bvec
        return carry

    lax.fori_loop(0, NCHUNK // NBUF, step, 0)

    pltpu.sync_copy(outp_v, pos_hbm.at[pl.ds(base, BPW)])
    pltpu.sync_copy(outn_v, neg_hbm.at[pl.ds(base, BPW)])


@jax.jit
def _gmf(utT, itT, wb, bb, lanes_u, lanes_i, starts_u, starts_i,
         posr_u, posr_p, posr_n):
    mesh = plsc.VectorSubcoreMesh(core_axis_name="c", subcore_axis_name="s")
    scan = pl.kernel(
        _scan_body,
        mesh=mesh,
        compiler_params=pltpu.CompilerParams(needs_layout_passes=False),
        out_type=[
            jax.ShapeDtypeStruct((BATCH, EMBED_DIM), jnp.float32),
            jax.ShapeDtypeStruct((NBATCH_I, EMBED_DIM), jnp.float32),
        ],
        scratch_types=[
            pltpu.VMEM((272,), jnp.int32),
            pltpu.VMEM((272,), jnp.int32),
            pltpu.VMEM((CAP_U,), jnp.int32),
            pltpu.VMEM((CAP_I,), jnp.int32),
            pltpu.VMEM((NBUF, EMBED_DIM, 128), jnp.float32),
            pltpu.VMEM((NBUF, EMBED_DIM, 128), jnp.float32),
            pltpu.VMEM((RING, EMBED_DIM), jnp.float32),
            pltpu.VMEM((RING, EMBED_DIM), jnp.float32),
            pltpu.SemaphoreType.DMA,
            pltpu.SemaphoreType.DMA,
            pltpu.SemaphoreType.DMA,
            pltpu.SemaphoreType.DMA,
        ],
    )
    scr_u, scr_i = scan(utT, itT, lanes_u, lanes_i, starts_u, starts_i)

    comp = pl.kernel(
        _compute_body,
        mesh=mesh,
        compiler_params=pltpu.CompilerParams(needs_layout_passes=False),
        out_type=[
            jax.ShapeDtypeStruct((BATCH,), jnp.float32),
            jax.ShapeDtypeStruct((BATCH,), jnp.float32),
        ],
        scratch_types=[
            pltpu.VMEM((BPW,), jnp.int32),
            pltpu.VMEM((BPW,), jnp.int32),
            pltpu.VMEM((BPW,), jnp.int32),
            pltpu.VMEM((EMBED_DIM,), jnp.float32),
            pltpu.VMEM((16,), jnp.float32),
            pltpu.VMEM((NBUF, CHUNK, EMBED_DIM), jnp.float32),
            pltpu.VMEM((NBUF, CHUNK, EMBED_DIM), jnp.float32),
            pltpu.VMEM((NBUF, CHUNK, EMBED_DIM), jnp.float32),
            pltpu.VMEM((CHUNK,), jnp.float32),
            pltpu.VMEM((CHUNK,), jnp.float32),
            pltpu.VMEM((BPW,), jnp.float32),
            pltpu.VMEM((BPW,), jnp.float32),
            pltpu.SemaphoreType.DMA,
            pltpu.SemaphoreType.DMA,
        ],
    )
    return comp(scr_u, scr_i, wb, bb, posr_u, posr_p, posr_n)


def _route(idx, n):
    order = jnp.argsort(idx)
    srt = idx[order]
    lanes = (srt % 128).astype(jnp.int32)
    lanes = jnp.concatenate([lanes, jnp.zeros((4096,), jnp.int32)])
    edges = (jnp.arange(NW * BLKW + 272, dtype=jnp.int32) * 128)
    starts = jnp.searchsorted(srt, edges).astype(jnp.int32)
    pos = jnp.zeros((n,), jnp.int32).at[order].set(
        jnp.arange(n, dtype=jnp.int32))
    return lanes, starts, pos


def kernel(user_table, item_table, W, b, users, pos_items, neg_items):
    utT = user_table.T
    itT = item_table.T
    wb = W.reshape(EMBED_DIM)
    bb = jnp.broadcast_to(b.reshape(1), (16,))
    users = users.astype(jnp.int32)
    cat = jnp.concatenate([pos_items.astype(jnp.int32),
                           neg_items.astype(jnp.int32)])
    lanes_u, starts_u, pos_u = _route(users, BATCH)
    lanes_i, starts_i, pos_i = _route(cat, NBATCH_I)
    pos, neg = _gmf(utT, itT, wb, bb, lanes_u, lanes_i, starts_u, starts_i,
                    pos_u, pos_i[:BATCH], pos_i[BATCH:])
    return pos, neg


# speedup vs baseline: 1.1042x; 1.1042x over previous
import functools

import jax
import jax.numpy as jnp
from jax import lax
from jax.experimental import pallas as pl
from jax.experimental.pallas import tpu as pltpu
from jax.experimental.pallas import tpu_sc as plsc

EMBED_DIM = 64
BATCH = 16384
NC = 2
NS = 16
NW = NC * NS
BPW = BATCH // NW
CHUNK = 16
NCHUNK = BPW // CHUNK
NBUF = 2
NROWS = 1000000
NBLK = (NROWS + 127) // 128
BLKW = 248
NBATCH_I = 2 * BATCH
CAP_U = 1536
CAP_I = 3072
RING = 16


def _scan_body(ut, it, lu_hbm, li_hbm, su_hbm, si_hbm,
               scr_u, scr_i,
               su_v, si_v, lu_v, li_v,
               slab_u, slab_i, ring_u, ring_i,
               semb0, semb1, semr_u, semr_i):
    c = lax.axis_index("c")
    s = lax.axis_index("s")
    w = s * NC + c
    q0 = w * BLKW

    pltpu.sync_copy(su_hbm.at[pl.ds(q0, 272)], su_v)
    pltpu.sync_copy(si_hbm.at[pl.ds(q0, 272)], si_v)

    s_u = su_v[pl.ds(0, 16)][0]
    a_u = pl.multiple_of((s_u // 8) * 8, 8)
    pltpu.sync_copy(lu_hbm.at[pl.ds(a_u, CAP_U)], lu_v)
    s_i = si_v[pl.ds(0, 16)][0]
    a_i = pl.multiple_of((s_i // 8) * 8, 8)
    pltpu.sync_copy(li_hbm.at[pl.ds(a_i, CAP_I)], li_v)

    sembs = [semb0, semb1]
    dims = [lax.iota(jnp.int32, 16) + 16 * cc for cc in range(4)]

    def fire(j, buf, in_range):
        q = q0 + j

        @pl.when(in_range & (q < NBLK))
        def _():
            off = pl.multiple_of(q * 128, 128)
            pltpu.async_copy(ut.at[:, pl.ds(off, 128)], slab_u.at[buf],
                             sembs[buf])
            pltpu.async_copy(it.at[:, pl.ds(off, 128)], slab_i.at[buf],
                             sembs[buf])

    def wait_slabs(j, buf):
        @pl.when(q0 + j < NBLK)
        def _():
            pltpu.make_async_copy(ut.at[:, pl.ds(0, 128)], slab_u.at[buf],
                                  sembs[buf]).wait()
            pltpu.make_async_copy(it.at[:, pl.ds(0, 128)], slab_i.at[buf],
                                  sembs[buf]).wait()

    fire(0, 0, True)

    def do_block(j, buf, carry):
        h_u, h_i = carry

        fire(j + 1, (buf + 1) % NBUF, j + 1 < BLKW)
        wait_slabs(j, buf)

        def handle(slab, lanes_v, a0, starts_v, ring, semr, scr, h0):
            p0 = starts_v[pl.ds(j, 16)][0]
            p1 = starts_v[pl.ds(j + 1, 16)][0]

            def hit(p, h):
                l = lanes_v[pl.ds(p - a0, 16)][0]
                lv = jnp.full((16,), l, jnp.int32)
                r = h % RING

                @pl.when(h >= RING)
                def _():
                    pltpu.make_async_copy(scr.at[0], ring.at[0], semr).wait()

                for cc in range(4):
                    ring[r, pl.ds(16 * cc, 16)] = plsc.load_gather(
                        slab.at[buf], [dims[cc], lv])
                pltpu.async_copy(ring.at[r], scr.at[p], semr)
                return h + 1

            return lax.fori_loop(p0, p1, hit, h0)

        h_u = handle(slab_u, lu_v, a_u, su_v, ring_u, semr_u, scr_u, h_u)
        h_i = handle(slab_i, li_v, a_i, si_v, ring_i, semr_i, scr_i, h_i)
        return h_u, h_i

    def block_pair(jj, carry):
        for buf in range(NBUF):
            carry = do_block(jj * NBUF + buf, buf, carry)
        return carry

    h_u, h_i = lax.fori_loop(0, BLKW // NBUF, block_pair,
                             (jnp.int32(0), jnp.int32(0)))

    for i in range(RING):
        @pl.when(i < h_u)
        def _():
            pltpu.make_async_copy(scr_u.at[0], ring_u.at[0], semr_u).wait()

        @pl.when(i < h_i)
        def _():
            pltpu.make_async_copy(scr_i.at[0], ring_i.at[0], semr_i).wait()


def _compute_body(scr_u, scr_i, w_hbm, b_hbm, pu_hbm, pp_hbm, pn_hbm,
                  pos_hbm, neg_hbm,
                  pu_v, pp_v, pn_v, w_v, b_v,
                  u_buf, p_buf, n_buf, tmp_p, tmp_n, outp_v, outn_v,
                  sem0, sem1):
    c = lax.axis_index("c")
    s = lax.axis_index("s")
    wid = s * NC + c
    base = wid * BPW

    pltpu.sync_copy(pu_hbm.at[pl.ds(base, BPW)], pu_v)
    pltpu.sync_copy(pp_hbm.at[pl.ds(base, BPW)], pp_v)
    pltpu.sync_copy(pn_hbm.at[pl.ds(base, BPW)], pn_v)
    pltpu.sync_copy(w_hbm, w_v)
    pltpu.sync_copy(b_hbm, b_v)

    sems = [sem0, sem1]
    bufs = [(u_buf, pu_v, scr_u), (p_buf, pp_v, scr_i), (n_buf, pn_v, scr_i)]

    def fire(k, slot):
        off = pl.multiple_of(k * CHUNK, CHUNK)
        for buf, idx_v, table in bufs:
            iv = idx_v[pl.ds(off, CHUNK)]
            for j in range(CHUNK):
                pltpu.async_copy(table.at[iv[j]], buf.at[slot, j], sems[slot])

    def drain(slot):
        for buf, idx_v, table in bufs:
            pltpu.make_async_copy(table.at[pl.ds(0, CHUNK)], buf.at[slot],
                                  sems[slot]).wait()

    lanes = lax.iota(jnp.int32, 16)
    last = lanes == 15
    wc = [w_v[pl.ds(cc * 16, 16)] for cc in range(EMBED_DIM // 16)]
    bvec = b_v[...]

    fire(0, 0)

    def step(kk, carry):
        for slot in range(NBUF):
            k = kk * NBUF + slot
            nxt = (slot + 1) % NBUF

            @pl.when(k + 1 < NCHUNK)
            def _():
                fire(k + 1, nxt)

            drain(slot)

            off = pl.multiple_of(k * CHUNK, CHUNK)
            for j in range(CHUNK):
                accp = None
                accn = None
                for cc in range(EMBED_DIM // 16):
                    sl = pl.ds(cc * 16, 16)
                    uw = u_buf[slot, j, sl] * wc[cc]
                    tp = uw * p_buf[slot, j, sl]
                    tn = uw * n_buf[slot, j, sl]
                    accp = tp if accp is None else accp + tp
                    accn = tn if accn is None else accn + tn
                idxj = jnp.full((16,), j, jnp.int32)
                plsc.store_scatter(tmp_p, [idxj], plsc.cumsum(accp), mask=last)
                plsc.store_scatter(tmp_n, [idxj], plsc.cumsum(accn), mask=last)
            outp_v[pl.ds(off, CHUNK)] = tmp_p[...] + bvec
            outn_v[pl.ds(off, CHUNK)] = tmp_n[...] + bvec
        return carry

    lax.fori_loop(0, NCHUNK // NBUF, step, 0)

    pltpu.sync_copy(outp_v, pos_hbm.at[pl.ds(base, BPW)])
    pltpu.sync_copy(outn_v, neg_hbm.at[pl.ds(base, BPW)])


@jax.jit
def _gmf(utT, itT, wb, bb, lanes_u, lanes_i, starts_u, starts_i,
         posr_u, posr_p, posr_n):
    mesh = plsc.VectorSubcoreMesh(core_axis_name="c", subcore_axis_name="s")
    scan = pl.kernel(
        _scan_body,
        mesh=mesh,
        compiler_params=pltpu.CompilerParams(needs_layout_passes=False),
        out_type=[
            jax.ShapeDtypeStruct((BATCH, EMBED_DIM), jnp.float32),
            jax.ShapeDtypeStruct((NBATCH_I, EMBED_DIM), jnp.float32),
        ],
        scratch_types=[
            pltpu.VMEM((272,), jnp.int32),
            pltpu.VMEM((272,), jnp.int32),
            pltpu.VMEM((CAP_U,), jnp.int32),
            pltpu.VMEM((CAP_I,), jnp.int32),
            pltpu.VMEM((NBUF, EMBED_DIM, 128), jnp.float32),
            pltpu.VMEM((NBUF, EMBED_DIM, 128), jnp.float32),
            pltpu.VMEM((RING, EMBED_DIM), jnp.float32),
            pltpu.VMEM((RING, EMBED_DIM), jnp.float32),
            pltpu.SemaphoreType.DMA,
            pltpu.SemaphoreType.DMA,
            pltpu.SemaphoreType.DMA,
            pltpu.SemaphoreType.DMA,
        ],
    )
    scr_u, scr_i = scan(utT, itT, lanes_u, lanes_i, starts_u, starts_i)

    comp = pl.kernel(
        _compute_body,
        mesh=mesh,
        compiler_params=pltpu.CompilerParams(needs_layout_passes=False),
        out_type=[
            jax.ShapeDtypeStruct((BATCH,), jnp.float32),
            jax.ShapeDtypeStruct((BATCH,), jnp.float32),
        ],
        scratch_types=[
            pltpu.VMEM((BPW,), jnp.int32),
            pltpu.VMEM((BPW,), jnp.int32),
            pltpu.VMEM((BPW,), jnp.int32),
            pltpu.VMEM((EMBED_DIM,), jnp.float32),
            pltpu.VMEM((16,), jnp.float32),
            pltpu.VMEM((NBUF, CHUNK, EMBED_DIM), jnp.float32),
            pltpu.VMEM((NBUF, CHUNK, EMBED_DIM), jnp.float32),
            pltpu.VMEM((NBUF, CHUNK, EMBED_DIM), jnp.float32),
            pltpu.VMEM((CHUNK,), jnp.float32),
            pltpu.VMEM((CHUNK,), jnp.float32),
            pltpu.VMEM((BPW,), jnp.float32),
            pltpu.VMEM((BPW,), jnp.float32),
            pltpu.SemaphoreType.DMA,
            pltpu.SemaphoreType.DMA,
        ],
    )
    return comp(scr_u, scr_i, wb, bb, posr_u, posr_p, posr_n)


def _route(idx, n):
    order = jnp.argsort(idx)
    srt = idx[order]
    lanes = (srt % 128).astype(jnp.int32)
    lanes = jnp.concatenate([lanes, jnp.zeros((4096,), jnp.int32)])
    edges = (jnp.arange(NW * BLKW + 272, dtype=jnp.int32) * 128)
    starts = jnp.searchsorted(srt, edges, method="sort").astype(jnp.int32)
    pos = jnp.zeros((n,), jnp.int32).at[order].set(
        jnp.arange(n, dtype=jnp.int32))
    return lanes, starts, pos


def kernel(user_table, item_table, W, b, users, pos_items, neg_items):
    utT = user_table.T
    itT = item_table.T
    wb = W.reshape(EMBED_DIM)
    bb = jnp.broadcast_to(b.reshape(1), (16,))
    users = users.astype(jnp.int32)
    cat = jnp.concatenate([pos_items.astype(jnp.int32),
                           neg_items.astype(jnp.int32)])
    lanes_u, starts_u, pos_u = _route(users, BATCH)
    lanes_i, starts_i, pos_i = _route(cat, NBATCH_I)
    pos, neg = _gmf(utT, itT, wb, bb, lanes_u, lanes_i, starts_u, starts_i,
                    pos_u, pos_i[:BATCH], pos_i[BATCH:])
    return pos, neg


# speedup vs baseline: 2.3115x; 2.0933x over previous
import functools

import jax
import jax.numpy as jnp
from jax import lax
from jax.experimental import pallas as pl
from jax.experimental.pallas import tpu as pltpu
from jax.experimental.pallas import tpu_sc as plsc

EMBED_DIM = 64
BATCH = 16384
NC = 2
NS = 16
NW = NC * NS
BPW = BATCH // NW
CHUNK = 16
NCHUNK = BPW // CHUNK
NBUF = 2
NROWS = 1000000
NBLK = (NROWS + 127) // 128
BLKW = 248
NBATCH_I = 2 * BATCH
CAP_U = 1536
CAP_I = 3072
RING = 16


def _scan_body(ut, it, sou_hbm, soi_hbm, swu_hbm, swi_hbm,
               scr_u, scr_i,
               swu_v, swi_v, seg_u, seg_i,
               slab_u, slab_i, ring_u, ring_i,
               semb0, semb1, semr_u, semr_i):
    c = lax.axis_index("c")
    s = lax.axis_index("s")
    w = s * NC + c
    q0 = w * BLKW

    pltpu.sync_copy(swu_hbm, swu_v)
    pltpu.sync_copy(swi_hbm, swi_v)
    s_u = swu_v[pl.ds(w, 16)][0]
    e_u = swu_v[pl.ds(w + 1, 16)][0]
    a_u = pl.multiple_of((s_u // 8) * 8, 8)
    pltpu.sync_copy(sou_hbm.at[pl.ds(a_u, CAP_U)], seg_u)
    s_i = swi_v[pl.ds(w, 16)][0]
    e_i = swi_v[pl.ds(w + 1, 16)][0]
    a_i = pl.multiple_of((s_i // 8) * 8, 8)
    pltpu.sync_copy(soi_hbm.at[pl.ds(a_i, CAP_I)], seg_i)

    sembs = [semb0, semb1]
    dims = [lax.iota(jnp.int32, 16) + 16 * cc for cc in range(4)]

    def fire(j, buf, in_range):
        q = q0 + j

        @pl.when(in_range & (q < NBLK))
        def _():
            off = pl.multiple_of(q * 128, 128)
            pltpu.async_copy(ut.at[:, pl.ds(off, 128)], slab_u.at[buf],
                             sembs[buf])
            pltpu.async_copy(it.at[:, pl.ds(off, 128)], slab_i.at[buf],
                             sembs[buf])

    def wait_slabs(j, buf):
        @pl.when(q0 + j < NBLK)
        def _():
            pltpu.make_async_copy(ut.at[:, pl.ds(0, 128)], slab_u.at[buf],
                                  sembs[buf]).wait()
            pltpu.make_async_copy(it.at[:, pl.ds(0, 128)], slab_i.at[buf],
                                  sembs[buf]).wait()

    fire(0, 0, True)

    def do_block(j, buf, carry):
        p_u, p_i = carry
        q = q0 + j

        fire(j + 1, (buf + 1) % NBUF, j + 1 < BLKW)
        wait_slabs(j, buf)

        def handle(slab, seg, a0, s0, e0, ring, semr, scr, p0):
            def cond(p):
                v = seg[pl.ds(p - a0, 16)][0]
                return (p < e0) & (v // 128 == q)

            def hit(p):
                v = seg[pl.ds(p - a0, 16)][0]
                lv = jnp.full((16,), v % 128, jnp.int32)
                r = p % RING

                @pl.when(p - s0 >= RING)
                def _():
                    pltpu.make_async_copy(scr.at[0], ring.at[0], semr).wait()

                for cc in range(4):
                    ring[r, pl.ds(16 * cc, 16)] = plsc.load_gather(
                        slab.at[buf], [dims[cc], lv])
                pltpu.async_copy(ring.at[r], scr.at[p], semr)
                return p + 1

            return lax.while_loop(cond, hit, p0)

        p_u = handle(slab_u, seg_u, a_u, s_u, e_u, ring_u, semr_u, scr_u, p_u)
        p_i = handle(slab_i, seg_i, a_i, s_i, e_i, ring_i, semr_i, scr_i, p_i)
        return p_u, p_i

    def block_pair(jj, carry):
        for buf in range(NBUF):
            carry = do_block(jj * NBUF + buf, buf, carry)
        return carry

    p_u, p_i = lax.fori_loop(0, BLKW // NBUF, block_pair, (s_u, s_i))
    h_u = p_u - s_u
    h_i = p_i - s_i

    for i in range(RING):
        @pl.when(i < h_u)
        def _():
            pltpu.make_async_copy(scr_u.at[0], ring_u.at[0], semr_u).wait()

        @pl.when(i < h_i)
        def _():
            pltpu.make_async_copy(scr_i.at[0], ring_i.at[0], semr_i).wait()


def _compute_body(scr_u, scr_i, w_hbm, b_hbm, pu_hbm, pp_hbm, pn_hbm,
                  pos_hbm, neg_hbm,
                  pu_v, pp_v, pn_v, w_v, b_v,
                  u_buf, p_buf, n_buf, tmp_p, tmp_n, outp_v, outn_v,
                  sem0, sem1):
    c = lax.axis_index("c")
    s = lax.axis_index("s")
    wid = s * NC + c
    base = wid * BPW

    pltpu.sync_copy(pu_hbm.at[pl.ds(base, BPW)], pu_v)
    pltpu.sync_copy(pp_hbm.at[pl.ds(base, BPW)], pp_v)
    pltpu.sync_copy(pn_hbm.at[pl.ds(base, BPW)], pn_v)
    pltpu.sync_copy(w_hbm, w_v)
    pltpu.sync_copy(b_hbm, b_v)

    sems = [sem0, sem1]
    bufs = [(u_buf, pu_v, scr_u), (p_buf, pp_v, scr_i), (n_buf, pn_v, scr_i)]

    def fire(k, slot):
        off = pl.multiple_of(k * CHUNK, CHUNK)
        for buf, idx_v, table in bufs:
            iv = idx_v[pl.ds(off, CHUNK)]
            for j in range(CHUNK):
                pltpu.async_copy(table.at[iv[j]], buf.at[slot, j], sems[slot])

    def drain(slot):
        for buf, idx_v, table in bufs:
            pltpu.make_async_copy(table.at[pl.ds(0, CHUNK)], buf.at[slot],
                                  sems[slot]).wait()

    lanes = lax.iota(jnp.int32, 16)
    last = lanes == 15
    wc = [w_v[pl.ds(cc * 16, 16)] for cc in range(EMBED_DIM // 16)]
    bvec = b_v[...]

    fire(0, 0)

    def step(kk, carry):
        for slot in range(NBUF):
            k = kk * NBUF + slot
            nxt = (slot + 1) % NBUF

            @pl.when(k + 1 < NCHUNK)
            def _():
                fire(k + 1, nxt)

            drain(slot)

            off = pl.multiple_of(k * CHUNK, CHUNK)
            for j in range(CHUNK):
                accp = None
                accn = None
                for cc in range(EMBED_DIM // 16):
                    sl = pl.ds(cc * 16, 16)
                    uw = u_buf[slot, j, sl] * wc[cc]
                    tp = uw * p_buf[slot, j, sl]
                    tn = uw * n_buf[slot, j, sl]
                    accp = tp if accp is None else accp + tp
                    accn = tn if accn is None else accn + tn
                idxj = jnp.full((16,), j, jnp.int32)
                plsc.store_scatter(tmp_p, [idxj], plsc.cumsum(accp), mask=last)
                plsc.store_scatter(tmp_n, [idxj], plsc.cumsum(accn), mask=last)
            outp_v[pl.ds(off, CHUNK)] = tmp_p[...] + bvec
            outn_v[pl.ds(off, CHUNK)] = tmp_n[...] + bvec
        return carry

    lax.fori_loop(0, NCHUNK // NBUF, step, 0)

    pltpu.sync_copy(outp_v, pos_hbm.at[pl.ds(base, BPW)])
    pltpu.sync_copy(outn_v, neg_hbm.at[pl.ds(base, BPW)])


@jax.jit
def _gmf(utT, itT, wb, bb, lanes_u, lanes_i, starts_u, starts_i,
         posr_u, posr_p, posr_n):
    mesh = plsc.VectorSubcoreMesh(core_axis_name="c", subcore_axis_name="s")
    scan = pl.kernel(
        _scan_body,
        mesh=mesh,
        compiler_params=pltpu.CompilerParams(needs_layout_passes=False),
        out_type=[
            jax.ShapeDtypeStruct((BATCH, EMBED_DIM), jnp.float32),
            jax.ShapeDtypeStruct((NBATCH_I, EMBED_DIM), jnp.float32),
        ],
        scratch_types=[
            pltpu.VMEM((48,), jnp.int32),
            pltpu.VMEM((48,), jnp.int32),
            pltpu.VMEM((CAP_U,), jnp.int32),
            pltpu.VMEM((CAP_I,), jnp.int32),
            pltpu.VMEM((NBUF, EMBED_DIM, 128), jnp.float32),
            pltpu.VMEM((NBUF, EMBED_DIM, 128), jnp.float32),
            pltpu.VMEM((RING, EMBED_DIM), jnp.float32),
            pltpu.VMEM((RING, EMBED_DIM), jnp.float32),
            pltpu.SemaphoreType.DMA,
            pltpu.SemaphoreType.DMA,
            pltpu.SemaphoreType.DMA,
            pltpu.SemaphoreType.DMA,
        ],
    )
    scr_u, scr_i = scan(utT, itT, lanes_u, lanes_i, starts_u, starts_i)

    comp = pl.kernel(
        _compute_body,
        mesh=mesh,
        compiler_params=pltpu.CompilerParams(needs_layout_passes=False),
        out_type=[
            jax.ShapeDtypeStruct((BATCH,), jnp.float32),
            jax.ShapeDtypeStruct((BATCH,), jnp.float32),
        ],
        scratch_types=[
            pltpu.VMEM((BPW,), jnp.int32),
            pltpu.VMEM((BPW,), jnp.int32),
            pltpu.VMEM((BPW,), jnp.int32),
            pltpu.VMEM((EMBED_DIM,), jnp.float32),
            pltpu.VMEM((16,), jnp.float32),
            pltpu.VMEM((NBUF, CHUNK, EMBED_DIM), jnp.float32),
            pltpu.VMEM((NBUF, CHUNK, EMBED_DIM), jnp.float32),
            pltpu.VMEM((NBUF, CHUNK, EMBED_DIM), jnp.float32),
            pltpu.VMEM((CHUNK,), jnp.float32),
            pltpu.VMEM((CHUNK,), jnp.float32),
            pltpu.VMEM((BPW,), jnp.float32),
            pltpu.VMEM((BPW,), jnp.float32),
            pltpu.SemaphoreType.DMA,
            pltpu.SemaphoreType.DMA,
        ],
    )
    return comp(scr_u, scr_i, wb, bb, posr_u, posr_p, posr_n)


def _route(idx, n, cap):
    order = jnp.argsort(idx)
    srt = idx[order].astype(jnp.int32)
    pos = jnp.argsort(order).astype(jnp.int32)
    edges = jnp.arange(48, dtype=jnp.int32) * (BLKW * 128)
    sw = jnp.sum(srt[None, :] < edges[:, None], axis=1).astype(jnp.int32)
    srt_pad = jnp.concatenate([srt, jnp.zeros((cap,), jnp.int32)])
    return srt_pad, sw, pos


def kernel(user_table, item_table, W, b, users, pos_items, neg_items):
    utT = user_table.T
    itT = item_table.T
    wb = W.reshape(EMBED_DIM)
    bb = jnp.broadcast_to(b.reshape(1), (16,))
    users = users.astype(jnp.int32)
    cat = jnp.concatenate([pos_items.astype(jnp.int32),
                           neg_items.astype(jnp.int32)])
    sorted_u, sw_u, pos_u = _route(users, BATCH, CAP_U)
    sorted_i, sw_i, pos_i = _route(cat, NBATCH_I, CAP_I)
    pos, neg = _gmf(utT, itT, wb, bb, sorted_u, sorted_i, sw_u, sw_i,
                    pos_u, pos_i[:BATCH], pos_i[BATCH:])
    return pos, neg


# speedup vs baseline: 2.9883x; 1.2928x over previous
import functools

import jax
import jax.numpy as jnp
from jax import lax
from jax.experimental import pallas as pl
from jax.experimental.pallas import tpu as pltpu
from jax.experimental.pallas import tpu_sc as plsc

EMBED_DIM = 64
BATCH = 16384
NC = 2
NS = 16
NW = NC * NS
BPW = BATCH // NW
CHUNK = 16
NCHUNK = BPW // CHUNK
NBUF = 2
NROWS = 1000000
NBLK = (NROWS + 127) // 128
BLKW = 248
NBATCH_I = 2 * BATCH
CAP_U = 1536
CAP_I = 3072
RING = 16
SBUF = 4


def _scan_body(ut, it, sou_hbm, soi_hbm, swu_hbm, swi_hbm,
               scr_u, scr_i,
               swu_v, swi_v, seg_u, seg_i,
               slab_u, slab_i, ring_u, ring_i,
               semb0, semb1, semb2, semb3, semr_u, semr_i):
    c = lax.axis_index("c")
    s = lax.axis_index("s")
    w = s * NC + c
    q0 = w * BLKW

    pltpu.sync_copy(swu_hbm, swu_v)
    pltpu.sync_copy(swi_hbm, swi_v)
    s_u = swu_v[pl.ds(w, 16)][0]
    e_u = swu_v[pl.ds(w + 1, 16)][0]
    a_u = pl.multiple_of((s_u // 8) * 8, 8)
    pltpu.sync_copy(sou_hbm.at[pl.ds(a_u, CAP_U)], seg_u)
    s_i = swi_v[pl.ds(w, 16)][0]
    e_i = swi_v[pl.ds(w + 1, 16)][0]
    a_i = pl.multiple_of((s_i // 8) * 8, 8)
    pltpu.sync_copy(soi_hbm.at[pl.ds(a_i, CAP_I)], seg_i)

    sembs = [semb0, semb1, semb2, semb3]
    dims = [lax.iota(jnp.int32, 16) + 16 * cc for cc in range(4)]

    def fire(j, buf, in_range):
        q = q0 + j

        @pl.when(in_range & (q < NBLK))
        def _():
            off = pl.multiple_of(q * 128, 128)
            pltpu.async_copy(ut.at[:, pl.ds(off, 128)], slab_u.at[buf],
                             sembs[buf])
            pltpu.async_copy(it.at[:, pl.ds(off, 128)], slab_i.at[buf],
                             sembs[buf])

    def wait_slabs(j, buf):
        @pl.when(q0 + j < NBLK)
        def _():
            pltpu.make_async_copy(ut.at[:, pl.ds(0, 128)], slab_u.at[buf],
                                  sembs[buf]).wait()
            pltpu.make_async_copy(it.at[:, pl.ds(0, 128)], slab_i.at[buf],
                                  sembs[buf]).wait()

    for jp in range(SBUF - 1):
        fire(jp, jp, True)

    def do_block(j, buf, carry):
        p_u, p_i = carry
        q = q0 + j

        fire(j + SBUF - 1, (buf + SBUF - 1) % SBUF, j + SBUF - 1 < BLKW)
        wait_slabs(j, buf)

        def handle(slab, seg, a0, s0, e0, ring, semr, scr, p0):
            def cond(p):
                v = seg[pl.ds(p - a0, 16)][0]
                return (p < e0) & (v // 128 == q)

            def hit(p):
                v = seg[pl.ds(p - a0, 16)][0]
                lv = jnp.full((16,), v % 128, jnp.int32)
                r = p % RING

                @pl.when(p - s0 >= RING)
                def _():
                    pltpu.make_async_copy(scr.at[0], ring.at[0], semr).wait()

                for cc in range(4):
                    ring[r, pl.ds(16 * cc, 16)] = plsc.load_gather(
                        slab.at[buf], [dims[cc], lv])
                pltpu.async_copy(ring.at[r], scr.at[p], semr)
                return p + 1

            return lax.while_loop(cond, hit, p0)

        p_u = handle(slab_u, seg_u, a_u, s_u, e_u, ring_u, semr_u, scr_u, p_u)
        p_i = handle(slab_i, seg_i, a_i, s_i, e_i, ring_i, semr_i, scr_i, p_i)
        return p_u, p_i

    def block_pair(jj, carry):
        for buf in range(SBUF):
            carry = do_block(jj * SBUF + buf, buf, carry)
        return carry

    p_u, p_i = lax.fori_loop(0, BLKW // SBUF, block_pair, (s_u, s_i))
    h_u = p_u - s_u
    h_i = p_i - s_i

    for i in range(RING):
        @pl.when(i < h_u)
        def _():
            pltpu.make_async_copy(scr_u.at[0], ring_u.at[0], semr_u).wait()

        @pl.when(i < h_i)
        def _():
            pltpu.make_async_copy(scr_i.at[0], ring_i.at[0], semr_i).wait()


def _compute_body(scr_u, scr_i, w_hbm, b_hbm, pu_hbm, pp_hbm, pn_hbm,
                  pos_hbm, neg_hbm,
                  pu_v, pp_v, pn_v, w_v, b_v,
                  u_buf, p_buf, n_buf, tmp_p, tmp_n, outp_v, outn_v,
                  sem0, sem1):
    c = lax.axis_index("c")
    s = lax.axis_index("s")
    wid = s * NC + c
    base = wid * BPW

    pltpu.sync_copy(pu_hbm.at[pl.ds(base, BPW)], pu_v)
    pltpu.sync_copy(pp_hbm.at[pl.ds(base, BPW)], pp_v)
    pltpu.sync_copy(pn_hbm.at[pl.ds(base, BPW)], pn_v)
    pltpu.sync_copy(w_hbm, w_v)
    pltpu.sync_copy(b_hbm, b_v)

    sems = [sem0, sem1]
    bufs = [(u_buf, pu_v, scr_u), (p_buf, pp_v, scr_i), (n_buf, pn_v, scr_i)]

    def fire(k, slot):
        off = pl.multiple_of(k * CHUNK, CHUNK)
        for buf, idx_v, table in bufs:
            iv = idx_v[pl.ds(off, CHUNK)]
            for j in range(CHUNK):
                pltpu.async_copy(table.at[iv[j]], buf.at[slot, j], sems[slot])

    def drain(slot):
        for buf, idx_v, table in bufs:
            pltpu.make_async_copy(table.at[pl.ds(0, CHUNK)], buf.at[slot],
                                  sems[slot]).wait()

    lanes = lax.iota(jnp.int32, 16)
    last = lanes == 15
    wc = [w_v[pl.ds(cc * 16, 16)] for cc in range(EMBED_DIM // 16)]
    bvec = b_v[...]

    fire(0, 0)

    def step(kk, carry):
        for slot in range(NBUF):
            k = kk * NBUF + slot
            nxt = (slot + 1) % NBUF

            @pl.when(k + 1 < NCHUNK)
            def _():
                fire(k + 1, nxt)

            drain(slot)

            off = pl.multiple_of(k * CHUNK, CHUNK)
            for j in range(CHUNK):
                accp = None
                accn = None
                for cc in range(EMBED_DIM // 16):
                    sl = pl.ds(cc * 16, 16)
                    uw = u_buf[slot, j, sl] * wc[cc]
                    tp = uw * p_buf[slot, j, sl]
                    tn = uw * n_buf[slot, j, sl]
                    accp = tp if accp is None else accp + tp
                    accn = tn if accn is None else accn + tn
                idxj = jnp.full((16,), j, jnp.int32)
                plsc.store_scatter(tmp_p, [idxj], plsc.cumsum(accp), mask=last)
                plsc.store_scatter(tmp_n, [idxj], plsc.cumsum(accn), mask=last)
            outp_v[pl.ds(off, CHUNK)] = tmp_p[...] + bvec
            outn_v[pl.ds(off, CHUNK)] = tmp_n[...] + bvec
        return carry

    lax.fori_loop(0, NCHUNK // NBUF, step, 0)

    pltpu.sync_copy(outp_v, pos_hbm.at[pl.ds(base, BPW)])
    pltpu.sync_copy(outn_v, neg_hbm.at[pl.ds(base, BPW)])


@jax.jit
def _gmf(utT, itT, wb, bb, lanes_u, lanes_i, starts_u, starts_i,
         posr_u, posr_p, posr_n):
    mesh = plsc.VectorSubcoreMesh(core_axis_name="c", subcore_axis_name="s")
    scan = pl.kernel(
        _scan_body,
        mesh=mesh,
        compiler_params=pltpu.CompilerParams(needs_layout_passes=False),
        out_type=[
            jax.ShapeDtypeStruct((BATCH, EMBED_DIM), jnp.float32),
            jax.ShapeDtypeStruct((NBATCH_I, EMBED_DIM), jnp.float32),
        ],
        scratch_types=[
            pltpu.VMEM((48,), jnp.int32),
            pltpu.VMEM((48,), jnp.int32),
            pltpu.VMEM((CAP_U,), jnp.int32),
            pltpu.VMEM((CAP_I,), jnp.int32),
            pltpu.VMEM((SBUF, EMBED_DIM, 128), jnp.float32),
            pltpu.VMEM((SBUF, EMBED_DIM, 128), jnp.float32),
            pltpu.VMEM((RING, EMBED_DIM), jnp.float32),
            pltpu.VMEM((RING, EMBED_DIM), jnp.float32),
            pltpu.SemaphoreType.DMA,
            pltpu.SemaphoreType.DMA,
            pltpu.SemaphoreType.DMA,
            pltpu.SemaphoreType.DMA,
            pltpu.SemaphoreType.DMA,
            pltpu.SemaphoreType.DMA,
        ],
    )
    scr_u, scr_i = scan(utT, itT, lanes_u, lanes_i, starts_u, starts_i)

    comp = pl.kernel(
        _compute_body,
        mesh=mesh,
        compiler_params=pltpu.CompilerParams(needs_layout_passes=False),
        out_type=[
            jax.ShapeDtypeStruct((BATCH,), jnp.float32),
            jax.ShapeDtypeStruct((BATCH,), jnp.float32),
        ],
        scratch_types=[
            pltpu.VMEM((BPW,), jnp.int32),
            pltpu.VMEM((BPW,), jnp.int32),
            pltpu.VMEM((BPW,), jnp.int32),
            pltpu.VMEM((EMBED_DIM,), jnp.float32),
            pltpu.VMEM((16,), jnp.float32),
            pltpu.VMEM((NBUF, CHUNK, EMBED_DIM), jnp.float32),
            pltpu.VMEM((NBUF, CHUNK, EMBED_DIM), jnp.float32),
            pltpu.VMEM((NBUF, CHUNK, EMBED_DIM), jnp.float32),
            pltpu.VMEM((CHUNK,), jnp.float32),
            pltpu.VMEM((CHUNK,), jnp.float32),
            pltpu.VMEM((BPW,), jnp.float32),
            pltpu.VMEM((BPW,), jnp.float32),
            pltpu.SemaphoreType.DMA,
            pltpu.SemaphoreType.DMA,
        ],
    )
    return comp(scr_u, scr_i, wb, bb, posr_u, posr_p, posr_n)


def _route(idx, n, cap):
    order = jnp.argsort(idx)
    srt = idx[order].astype(jnp.int32)
    pos = jnp.argsort(order).astype(jnp.int32)
    edges = jnp.arange(48, dtype=jnp.int32) * (BLKW * 128)
    sw = jnp.sum(srt[None, :] < edges[:, None], axis=1).astype(jnp.int32)
    srt_pad = jnp.concatenate([srt, jnp.zeros((cap,), jnp.int32)])
    return srt_pad, sw, pos


def kernel(user_table, item_table, W, b, users, pos_items, neg_items):
    utT = user_table.T
    itT = item_table.T
    wb = W.reshape(EMBED_DIM)
    bb = jnp.broadcast_to(b.reshape(1), (16,))
    users = users.astype(jnp.int32)
    cat = jnp.concatenate([pos_items.astype(jnp.int32),
                           neg_items.astype(jnp.int32)])
    sorted_u, sw_u, pos_u = _route(users, BATCH, CAP_U)
    sorted_i, sw_i, pos_i = _route(cat, NBATCH_I, CAP_I)
    pos, neg = _gmf(utT, itT, wb, bb, sorted_u, sorted_i, sw_u, sw_i,
                    pos_u, pos_i[:BATCH], pos_i[BATCH:])
    return pos, neg
